# baseline (speedup 1.0000x reference)
import jax
import jax.numpy as jnp
from jax.experimental import pallas as pl
from jax.experimental.pallas import tpu as pltpu

NUM_RELATIONS = 4
RDIMS = 32
OUTPUT_DIM = NUM_RELATIONS * RDIMS
TB = 8


def _coref_gru_kernel(x_ref, m_ref, code_ref,
                      w_ref, u_ref, b_ref, watt_ref,
                      out_ref, mem_out_ref, agg_ref,
                      h_scr, m_scr):
    t = pl.program_id(0)

    @pl.when(t == 0)
    def _init():
        h_scr[...] = jnp.zeros_like(h_scr)
        m_scr[...] = jnp.zeros_like(m_scr)

    B = x_ref.shape[0]
    D = x_ref.shape[2]
    NB = m_scr.shape[1]
    L = NUM_RELATIONS * RDIMS

    bias = b_ref[0, :]

    lane = jax.lax.broadcasted_iota(jnp.int32, (NUM_RELATIONS, L), 1)
    row = jax.lax.broadcasted_iota(jnp.int32, (NUM_RELATIONS, L), 0)
    K = (lane // RDIMS == row).astype(jnp.float32)
    gl = jax.lax.broadcasted_iota(jnp.int32, (L, RDIMS), 0)
    gd = jax.lax.broadcasted_iota(jnp.int32, (L, RDIMS), 1)
    G = (gl % RDIMS == gd).astype(jnp.float32)

    def expand(q):
        return jax.lax.dot_general(q, K, (((2,), (0,)), ((), ())),
                                   preferred_element_type=jnp.float32)

    mprev = m_scr[...]
    hprev = h_scr[...]

    for j in range(TB):
        x = x_ref[:, j, :]
        code = code_ref[:, j, :]
        mgate = m_ref[:, j, :]
        ri2 = code & 3
        ro2 = (code >> 2) & 3
        ei2 = ((code >> 4) & 1).astype(jnp.float32)
        eo2 = (code >> 5).astype(jnp.float32)

        xw = jax.lax.dot_general(x, w_ref[...], (((1,), (0,)), ((), ())),
                                 preferred_element_type=jnp.float32)
        sc = jax.lax.dot_general(x, watt_ref[...], (((1,), (1,)), ((), ())),
                                 preferred_element_type=jnp.float32)

        actvs = jnp.zeros_like(ei2)
        for r in range(NUM_RELATIONS):
            actvs = jnp.where(ri2 == r, sc[:, r:r + 1], actvs)

        am = jnp.exp(actvs) * ei2
        denom = jnp.sum(am, axis=1, keepdims=True)
        alphas = (am / denom).reshape(B, NB, NUM_RELATIONS)
        ri = ri2.reshape(B, NB, NUM_RELATIONS)
        ro = ro2.reshape(B, NB, NUM_RELATIONS)

        mem_parts = []
        agg_parts = []
        for r in range(NUM_RELATIONS):
            wr = jnp.where(ri == r, alphas, 0.0)
            part = jnp.sum(expand(wr) * mprev, axis=1)
            mem_parts.append(
                jax.lax.dot_general(part, G, (((1,), (0,)), ((), ())),
                                    preferred_element_type=jnp.float32))
            agg_parts.append(jnp.sum(wr, axis=(1, 2), keepdims=True))
        prev = jnp.concatenate(mem_parts, axis=1)
        aggs = jnp.concatenate(agg_parts, axis=2)

        hid = jax.lax.dot_general(prev, u_ref[...], (((1,), (0,)), ((), ())),
                                  preferred_element_type=jnp.float32)

        g = jax.nn.sigmoid(xw + hid + bias)
        ht = jnp.tanh(xw + g * hid + bias)
        hnew = (1.0 - g) * prev + g * ht

        wgt = (mgate * eo2).reshape(B, NB, NUM_RELATIONS)
        mout = mprev * (1.0 - expand(wgt))
        for r in range(NUM_RELATIONS):
            cr = jnp.where(ro == r, wgt, 0.0)
            hr = hnew[:, r * RDIMS:(r + 1) * RDIMS]
            tile = jnp.concatenate([hr] * NUM_RELATIONS, axis=1)
            tile3 = jax.lax.broadcast_in_dim(tile, (B, 1, L), (0, 2))
            mout = mout + expand(cr) * tile3

        hout = (1.0 - mgate) * hprev + mgate * hnew

        out_ref[:, j, :] = hout
        mem_out_ref[:, j, :, :] = mout
        agg_ref[:, j, :] = aggs[:, 0, :]
        hprev = hout
        mprev = mout

    h_scr[...] = hprev
    m_scr[...] = mprev


@jax.jit
def kernel(X, M, Ei, Eo, Ri, Ro, W, U, b, Watt):
    B, T, D = X.shape
    N = Ri.shape[2]
    NB = N // NUM_RELATIONS
    L = NUM_RELATIONS * RDIMS

    code = Ri + (Ro << 2) + (Ei << 4) + (Eo << 5)
    M3 = M.reshape(B, T, 1)
    b2 = b.reshape(1, OUTPUT_DIM)

    tspec = lambda blk: pl.BlockSpec(blk, lambda t: (0, t, 0))
    full_spec = lambda shp: pl.BlockSpec(shp, lambda t: tuple(0 for _ in shp))

    grid = (T + TB - 1) // TB

    outs, mems, aggs = pl.pallas_call(
        _coref_gru_kernel,
        grid=(grid,),
        in_specs=[
            tspec((B, TB, D)),
            tspec((B, TB, 1)),
            tspec((B, TB, N)),
            full_spec((D, OUTPUT_DIM)),
            full_spec((OUTPUT_DIM, OUTPUT_DIM)),
            full_spec((1, OUTPUT_DIM)),
            full_spec((NUM_RELATIONS, D)),
        ],
        out_specs=[
            tspec((B, TB, OUTPUT_DIM)),
            pl.BlockSpec((B, TB, NB, L), lambda t: (0, t, 0, 0)),
            tspec((B, TB, NUM_RELATIONS)),
        ],
        out_shape=[
            jax.ShapeDtypeStruct((B, T, OUTPUT_DIM), jnp.float32),
            jax.ShapeDtypeStruct((B, T, NB, L), jnp.float32),
            jax.ShapeDtypeStruct((B, T, NUM_RELATIONS), jnp.float32),
        ],
        scratch_shapes=[
            pltpu.VMEM((B, OUTPUT_DIM), jnp.float32),
            pltpu.VMEM((B, NB, L), jnp.float32),
        ],
    )(X, M3, code, W, U, b2, Watt)

    return (outs, mems.reshape(B, T, N, RDIMS), aggs)


# speedup vs baseline: 3.0050x; 3.0050x over previous
import jax
import jax.numpy as jnp
from jax.experimental import pallas as pl
from jax.experimental.pallas import tpu as pltpu
from jax.experimental.layout import Format, Layout, with_layout_constraint

NUM_RELATIONS = 4
RDIMS = 32
OUTPUT_DIM = NUM_RELATIONS * RDIMS
TB = 4


def _coref_gru_kernel(x_ref, m_ref, code_ref,
                      w_ref, u_ref, b_ref, watt_ref,
                      out_ref, mem_out_ref, agg_ref,
                      h_scr, m_scr):
    t = pl.program_id(0)

    @pl.when(t == 0)
    def _init():
        h_scr[...] = jnp.zeros_like(h_scr)
        m_scr[...] = jnp.zeros_like(m_scr)

    B = x_ref.shape[1]
    D = x_ref.shape[2]
    NB = m_scr.shape[1]
    L = NUM_RELATIONS * RDIMS

    bias = b_ref[0, :]

    xall = x_ref[...].reshape(TB * B, D)
    xwall = jax.lax.dot_general(xall, w_ref[...], (((1,), (0,)), ((), ())),
                                preferred_element_type=jnp.float32
                                ).reshape(TB, B, OUTPUT_DIM)
    scall = jax.lax.dot_general(xall, watt_ref[...], (((1,), (1,)), ((), ())),
                                preferred_element_type=jnp.float32
                                ).reshape(TB, B, NUM_RELATIONS)

    lane = jax.lax.broadcasted_iota(jnp.int32, (NUM_RELATIONS, L), 1)
    row = jax.lax.broadcasted_iota(jnp.int32, (NUM_RELATIONS, L), 0)
    K = (lane // RDIMS == row).astype(jnp.float32)
    gl = jax.lax.broadcasted_iota(jnp.int32, (L, RDIMS), 0)
    gd = jax.lax.broadcasted_iota(jnp.int32, (L, RDIMS), 1)
    G = (gl % RDIMS == gd).astype(jnp.float32)

    def expand(q):
        return jax.lax.dot_general(q, K, (((2,), (0,)), ((), ())),
                                   preferred_element_type=jnp.float32)

    mprev = m_scr[...]
    hprev = h_scr[...]

    for j in range(TB):
        code = code_ref[j]
        ri2 = code & 3
        ro2 = (code >> 2) & 3
        ei2 = ((code >> 4) & 1).astype(jnp.float32)
        eo2 = (code >> 5).astype(jnp.float32)
        mgate = m_ref[j]
        xw = xwall[j]
        sc = scall[j]

        actvs = jnp.zeros_like(ei2)
        for r in range(NUM_RELATIONS):
            actvs = jnp.where(ri2 == r, sc[:, r:r + 1], actvs)

        am = jnp.exp(actvs) * ei2
        denom = jnp.sum(am, axis=1, keepdims=True)
        alphas = (am / denom).reshape(B, NB, NUM_RELATIONS)
        ri = ri2.reshape(B, NB, NUM_RELATIONS)
        ro = ro2.reshape(B, NB, NUM_RELATIONS)

        mem_parts = []
        agg_parts = []
        for r in range(NUM_RELATIONS):
            wr = jnp.where(ri == r, alphas, 0.0)
            part = jnp.sum(expand(wr) * mprev, axis=1)
            mem_parts.append(
                jax.lax.dot_general(part, G, (((1,), (0,)), ((), ())),
                                    preferred_element_type=jnp.float32))
            agg_parts.append(jnp.sum(wr, axis=(1, 2), keepdims=True))
        prev = jnp.concatenate(mem_parts, axis=1)
        aggs = jnp.concatenate(agg_parts, axis=2)

        hid = jax.lax.dot_general(prev, u_ref[...], (((1,), (0,)), ((), ())),
                                  preferred_element_type=jnp.float32)

        g = jax.nn.sigmoid(xw + hid + bias)
        ht = jnp.tanh(xw + g * hid + bias)
        hnew = (1.0 - g) * prev + g * ht

        wgt = (mgate * eo2).reshape(B, NB, NUM_RELATIONS)
        mout = mprev * (1.0 - expand(wgt))
        for r in range(NUM_RELATIONS):
            cr = jnp.where(ro == r, wgt, 0.0)
            hr = hnew[:, r * RDIMS:(r + 1) * RDIMS]
            tile = jnp.concatenate([hr] * NUM_RELATIONS, axis=1)
            tile3 = jax.lax.broadcast_in_dim(tile, (B, 1, L), (0, 2))
            mout = mout + expand(cr) * tile3

        hout = (1.0 - mgate) * hprev + mgate * hnew

        out_ref[j] = hout
        mem_out_ref[:, j, :, :] = mout
        agg_ref[j] = aggs[:, 0, :]
        hprev = hout
        mprev = mout

    h_scr[...] = hprev
    m_scr[...] = mprev


def _kernel_impl(X, M, Ei, Eo, Ri, Ro, W, U, b, Watt):
    B, T, D = X.shape
    N = Ri.shape[2]
    NB = N // NUM_RELATIONS
    L = NUM_RELATIONS * RDIMS

    Xt = jnp.transpose(X, (1, 0, 2))
    Mt = jnp.transpose(M, (1, 0)).reshape(T, B, 1)
    code = Ri + (Ro << 2) + (Ei << 4) + (Eo << 5)
    codeT = jnp.transpose(code, (1, 0, 2))
    b2 = b.reshape(1, OUTPUT_DIM)

    tspec = lambda blk: pl.BlockSpec(blk, lambda t: (t, 0, 0))
    full_spec = lambda shp: pl.BlockSpec(shp, lambda t: tuple(0 for _ in shp))

    outs, mems, aggs = pl.pallas_call(
        _coref_gru_kernel,
        grid=(T // TB,),
        in_specs=[
            tspec((TB, B, D)),
            tspec((TB, B, 1)),
            tspec((TB, B, N)),
            full_spec((D, OUTPUT_DIM)),
            full_spec((OUTPUT_DIM, OUTPUT_DIM)),
            full_spec((1, OUTPUT_DIM)),
            full_spec((NUM_RELATIONS, D)),
        ],
        out_specs=[
            tspec((TB, B, OUTPUT_DIM)),
            pl.BlockSpec((B, TB, NB, L), lambda t: (0, t, 0, 0)),
            tspec((TB, B, NUM_RELATIONS)),
        ],
        out_shape=[
            jax.ShapeDtypeStruct((T, B, OUTPUT_DIM), jnp.float32),
            jax.ShapeDtypeStruct((B, T, NB, L), jnp.float32),
            jax.ShapeDtypeStruct((T, B, NUM_RELATIONS), jnp.float32),
        ],
        scratch_shapes=[
            pltpu.VMEM((B, OUTPUT_DIM), jnp.float32),
            pltpu.VMEM((B, NB, L), jnp.float32),
        ],
    )(Xt, Mt, codeT, W, U, b2, Watt)

    mems4 = mems.reshape(B, T, N, RDIMS)
    mems4 = with_layout_constraint(
        mems4, Layout(major_to_minor=(0, 2, 3, 1), tiling=((8, 128),)))
    return (jnp.transpose(outs, (1, 0, 2)),
            mems4,
            jnp.transpose(aggs, (1, 0, 2)))


_MEMS_LAYOUT = Layout(major_to_minor=(0, 2, 3, 1), tiling=((8, 128),))
_jit_cache = {}


def kernel(X, M, Ei, Eo, Ri, Ro, W, U, b, Watt):
    dev = getattr(X, "device", None)
    fn = _jit_cache.get(dev)
    if fn is None:
        if dev is None:
            fn = jax.jit(_kernel_impl)
        else:
            sh = jax.sharding.SingleDeviceSharding(dev)
            fn = jax.jit(_kernel_impl,
                         out_shardings=(sh, Format(_MEMS_LAYOUT, sh), sh))
        _jit_cache[dev] = fn
    return fn(X, M, Ei, Eo, Ri, Ro, W, U, b, Watt)


# speedup vs baseline: 9.2090x; 3.0646x over previous
import jax
import jax.numpy as jnp
from jax.experimental import pallas as pl
from jax.experimental.pallas import tpu as pltpu

NUM_RELATIONS = 4
RDIMS = 32
OUTPUT_DIM = NUM_RELATIONS * RDIMS
TB = 4


def _coref_gru_kernel(x_ref, m_ref, code_ref,
                      w_ref, u_ref, b_ref, watt_ref,
                      out_ref, mem_out_ref, agg_ref,
                      h_scr, m_scr):
    t = pl.program_id(0)

    @pl.when(t == 0)
    def _init():
        h_scr[...] = jnp.zeros_like(h_scr)
        m_scr[...] = jnp.zeros_like(m_scr)

    B = x_ref.shape[1]
    D = x_ref.shape[2]
    N = code_ref.shape[2]

    bias = b_ref[0, :]

    xall = x_ref[...].reshape(TB * B, D)
    xwall = jax.lax.dot_general(xall, w_ref[...], (((1,), (0,)), ((), ())),
                                preferred_element_type=jnp.float32
                                ).reshape(TB, B, OUTPUT_DIM)
    scall = jax.lax.dot_general(xall, watt_ref[...], (((1,), (1,)), ((), ())),
                                preferred_element_type=jnp.float32
                                ).reshape(TB, B, NUM_RELATIONS)

    def lead(q):
        return jax.lax.broadcast_in_dim(q, (B, 1, N), (0, 2))

    mprev = m_scr[...]
    hprev = h_scr[...]

    for j in range(TB):
        code = code_ref[j]
        ri2 = code & 3
        ro2 = (code >> 2) & 3
        ei2 = ((code >> 4) & 1).astype(jnp.float32)
        eo2 = (code >> 5).astype(jnp.float32)
        mgate = m_ref[j]
        xw = xwall[j]
        sc = scall[j]

        actvs = jnp.zeros_like(ei2)
        for r in range(NUM_RELATIONS):
            actvs = jnp.where(ri2 == r, sc[:, r:r + 1], actvs)

        am = jnp.exp(actvs) * ei2
        denom = jnp.sum(am, axis=1, keepdims=True)
        alphas = am / denom

        mem_parts = []
        agg_parts = []
        for r in range(NUM_RELATIONS):
            wr = jnp.where(ri2 == r, alphas, 0.0)
            mem_parts.append(jnp.sum(lead(wr) * mprev, axis=2))
            agg_parts.append(jnp.sum(wr, axis=1, keepdims=True))
        prev = jnp.concatenate(mem_parts, axis=1)
        aggs = jnp.concatenate(agg_parts, axis=1)

        hid = jax.lax.dot_general(prev, u_ref[...], (((1,), (0,)), ((), ())),
                                  preferred_element_type=jnp.float32)

        g = jax.nn.sigmoid(xw + hid + bias)
        ht = jnp.tanh(xw + g * hid + bias)
        hnew = (1.0 - g) * prev + g * ht

        wgt = mgate * eo2
        mout = mprev * (1.0 - lead(wgt))
        for r in range(NUM_RELATIONS):
            c3 = lead(jnp.where(ro2 == r, wgt, 0.0))
            h3 = jax.lax.broadcast_in_dim(
                hnew[:, r * RDIMS:(r + 1) * RDIMS], (B, RDIMS, 1), (0, 1))
            mout = mout + c3 * h3
        hout = (1.0 - mgate) * hprev + mgate * hnew

        out_ref[j] = hout
        mem_out_ref[:, j, :, :] = mout
        agg_ref[j] = aggs
        hprev = hout
        mprev = mout

    h_scr[...] = hprev
    m_scr[...] = mprev


def _kernel_impl(X, M, Ei, Eo, Ri, Ro, W, U, b, Watt):
    B, T, D = X.shape
    N = Ri.shape[2]

    Xt = jnp.transpose(X, (1, 0, 2))
    Mt = jnp.transpose(M, (1, 0)).reshape(T, B, 1)
    code = Ri + (Ro << 2) + (Ei << 4) + (Eo << 5)
    codeT = jnp.transpose(code, (1, 0, 2))
    b2 = b.reshape(1, OUTPUT_DIM)

    tspec = lambda blk: pl.BlockSpec(blk, lambda t: (t, 0, 0))
    full_spec = lambda shp: pl.BlockSpec(shp, lambda t: tuple(0 for _ in shp))

    outs, mems, aggs = pl.pallas_call(
        _coref_gru_kernel,
        grid=(T // TB,),
        in_specs=[
            tspec((TB, B, D)),
            tspec((TB, B, 1)),
            tspec((TB, B, N)),
            full_spec((D, OUTPUT_DIM)),
            full_spec((OUTPUT_DIM, OUTPUT_DIM)),
            full_spec((1, OUTPUT_DIM)),
            full_spec((NUM_RELATIONS, D)),
        ],
        out_specs=[
            tspec((TB, B, OUTPUT_DIM)),
            pl.BlockSpec((B, TB, RDIMS, N), lambda t: (0, t, 0, 0)),
            tspec((TB, B, NUM_RELATIONS)),
        ],
        out_shape=[
            jax.ShapeDtypeStruct((T, B, OUTPUT_DIM), jnp.float32),
            jax.ShapeDtypeStruct((B, T, RDIMS, N), jnp.float32),
            jax.ShapeDtypeStruct((T, B, NUM_RELATIONS), jnp.float32),
        ],
        scratch_shapes=[
            pltpu.VMEM((B, OUTPUT_DIM), jnp.float32),
            pltpu.VMEM((B, RDIMS, N), jnp.float32),
        ],
    )(Xt, Mt, codeT, W, U, b2, Watt)

    return (jnp.transpose(outs, (1, 0, 2)),
            jnp.transpose(mems, (0, 1, 3, 2)),
            jnp.transpose(aggs, (1, 0, 2)))


kernel = jax.jit(_kernel_impl)


# speedup vs baseline: 11.1848x; 1.2146x over previous
import jax
import jax.numpy as jnp
from jax.experimental import pallas as pl
from jax.experimental.pallas import tpu as pltpu

NUM_RELATIONS = 4
RDIMS = 32
OUTPUT_DIM = NUM_RELATIONS * RDIMS
TB = 10


def _coref_gru_kernel(x_ref, m_ref, code_ref,
                      w_ref, u_ref, b_ref, watt_ref,
                      out_ref, mem_out_ref, agg_ref,
                      h_scr, m_scr):
    t = pl.program_id(0)

    @pl.when(t == 0)
    def _init():
        h_scr[...] = jnp.zeros_like(h_scr)
        m_scr[...] = jnp.zeros_like(m_scr)

    B = x_ref.shape[1]
    D = x_ref.shape[2]
    N = code_ref.shape[2]

    bias = b_ref[0, :]

    xall = x_ref[...].reshape(TB * B, D)
    xwall = jax.lax.dot_general(xall, w_ref[...], (((1,), (0,)), ((), ())),
                                preferred_element_type=jnp.float32
                                ).reshape(TB, B, OUTPUT_DIM)
    scall = jax.lax.dot_general(xall, watt_ref[...], (((1,), (1,)), ((), ())),
                                preferred_element_type=jnp.float32
                                ).reshape(TB, B, NUM_RELATIONS)

    def lead(q):
        return jax.lax.broadcast_in_dim(q, (B, 1, N), (0, 2))

    codes = code_ref[...]
    ri3 = codes & 3
    ro3 = (codes >> 2) & 3
    ei3 = ((codes >> 4) & 1).astype(jnp.float32)
    eo3 = (codes >> 5).astype(jnp.float32)
    actvs3 = jnp.zeros_like(ei3)
    for r in range(NUM_RELATIONS):
        actvs3 = jnp.where(ri3 == r, scall[:, :, r:r + 1], actvs3)
    am3 = jnp.exp(actvs3) * ei3
    denom3 = jnp.sum(am3, axis=2, keepdims=True)
    alphas3 = am3 / denom3
    wgt3 = m_ref[...] * eo3

    mprev = m_scr[...]
    hprev = h_scr[...]

    for j in range(TB):
        ri2 = ri3[j]
        ro2 = ro3[j]
        alphas = alphas3[j]
        mgate = m_ref[j]
        xw = xwall[j]

        mem_parts = []
        agg_parts = []
        for r in range(NUM_RELATIONS):
            wr = jnp.where(ri2 == r, alphas, 0.0)
            mem_parts.append(jnp.sum(lead(wr) * mprev, axis=2))
            agg_parts.append(jnp.sum(wr, axis=1, keepdims=True))
        prev = jnp.concatenate(mem_parts, axis=1)
        aggs = jnp.concatenate(agg_parts, axis=1)

        hid = jax.lax.dot_general(prev, u_ref[...], (((1,), (0,)), ((), ())),
                                  preferred_element_type=jnp.float32)

        g = jax.nn.sigmoid(xw + hid + bias)
        ht = jnp.tanh(xw + g * hid + bias)
        hnew = (1.0 - g) * prev + g * ht

        wgt = wgt3[j]
        mout = mprev * (1.0 - lead(wgt))
        hcol = jax.lax.broadcast_in_dim(hnew, (B, OUTPUT_DIM, 1), (0, 1))
        for r in range(NUM_RELATIONS):
            c3 = lead(jnp.where(ro2 == r, wgt, 0.0))
            h3 = hcol[:, r * RDIMS:(r + 1) * RDIMS, :]
            mout = mout + c3 * h3
        hout = (1.0 - mgate) * hprev + mgate * hnew

        out_ref[j] = hout
        mem_out_ref[:, j, :, :] = mout
        agg_ref[j] = aggs
        hprev = hout
        mprev = mout

    h_scr[...] = hprev
    m_scr[...] = mprev


def _kernel_impl(X, M, Ei, Eo, Ri, Ro, W, U, b, Watt):
    B, T, D = X.shape
    N = Ri.shape[2]

    Xt = jnp.transpose(X, (1, 0, 2))
    Mt = jnp.transpose(M, (1, 0)).reshape(T, B, 1)
    code = Ri + (Ro << 2) + (Ei << 4) + (Eo << 5)
    codeT = jnp.transpose(code, (1, 0, 2))
    b2 = b.reshape(1, OUTPUT_DIM)

    tspec = lambda blk: pl.BlockSpec(blk, lambda t: (t, 0, 0))
    full_spec = lambda shp: pl.BlockSpec(shp, lambda t: tuple(0 for _ in shp))

    outs, mems, aggs = pl.pallas_call(
        _coref_gru_kernel,
        grid=(T // TB,),
        in_specs=[
            tspec((TB, B, D)),
            tspec((TB, B, 1)),
            tspec((TB, B, N)),
            full_spec((D, OUTPUT_DIM)),
            full_spec((OUTPUT_DIM, OUTPUT_DIM)),
            full_spec((1, OUTPUT_DIM)),
            full_spec((NUM_RELATIONS, D)),
        ],
        out_specs=[
            tspec((TB, B, OUTPUT_DIM)),
            pl.BlockSpec((B, TB, RDIMS, N), lambda t: (0, t, 0, 0)),
            tspec((TB, B, NUM_RELATIONS)),
        ],
        out_shape=[
            jax.ShapeDtypeStruct((T, B, OUTPUT_DIM), jnp.float32),
            jax.ShapeDtypeStruct((B, T, RDIMS, N), jnp.float32),
            jax.ShapeDtypeStruct((T, B, NUM_RELATIONS), jnp.float32),
        ],
        scratch_shapes=[
            pltpu.VMEM((B, OUTPUT_DIM), jnp.float32),
            pltpu.VMEM((B, RDIMS, N), jnp.float32),
        ],
    )(Xt, Mt, codeT, W, U, b2, Watt)

    return (jnp.transpose(outs, (1, 0, 2)),
            jnp.transpose(mems, (0, 1, 3, 2)),
            jnp.transpose(aggs, (1, 0, 2)))


kernel = jax.jit(_kernel_impl)
